# Initial kernel scaffold; baseline (speedup 1.0000x reference)
import jax
import jax.numpy as jnp
from jax.experimental import pallas as pl


def kernel(atom_types, table):
    raise NotImplementedError("write your pallas kernel here")


# speedup vs baseline: 1.1640x; 1.1640x over previous
import functools

import jax
import jax.numpy as jnp
from jax import lax
from jax.experimental import pallas as pl
from jax.experimental.pallas import tpu as pltpu
from jax.experimental.pallas import tpu_sc as plsc

NC = 2
NS = 16
NW = NC * NS
D = 80
B = 64 * 4096
BPW = B // NW
C = 512
NCHUNK = BPW // C


def _emb_kernel(idx_hbm, table_hbm, out_hbm, idx_v, rows_v, gsem, osem0, osem1):
    wid = lax.axis_index("s") * NC + lax.axis_index("c")
    base = wid * BPW
    osems = (osem0, osem1)
    copies = [None, None]
    for c in range(NCHUNK):
        b = c % 2
        if copies[b] is not None:
            copies[b].wait()
        pltpu.sync_copy(idx_hbm.at[pl.ds(base + c * C, C)], idx_v.at[b])
        pltpu.async_copy(table_hbm.at[idx_v.at[b]], rows_v.at[b], gsem).wait()
        copies[b] = pltpu.async_copy(
            rows_v.at[b], out_hbm.at[pl.ds(base + c * C, C)], osems[b])
    for cp in copies:
        if cp is not None:
            cp.wait()


@functools.partial(jax.jit, static_argnames=())
def _emb(idx_flat, table):
    mesh = plsc.VectorSubcoreMesh(core_axis_name="c", subcore_axis_name="s")
    run = pl.kernel(
        _emb_kernel,
        out_type=jax.ShapeDtypeStruct((B, D), jnp.float32),
        mesh=mesh,
        scratch_types=[
            pltpu.VMEM((2, C), jnp.int32),
            pltpu.VMEM((2, C, D), jnp.float32),
            pltpu.SemaphoreType.DMA,
            pltpu.SemaphoreType.DMA,
            pltpu.SemaphoreType.DMA,
        ],
        compiler_params=pltpu.CompilerParams(use_tc_tiling_on_sc=False),
    )
    return run(idx_flat, table)


def kernel(atom_types, table):
    idx_flat = atom_types.reshape(-1).astype(jnp.int32)
    out = _emb(idx_flat, table)
    return out.reshape(atom_types.shape + (D,))


# speedup vs baseline: 2.5251x; 2.1695x over previous
import jax
import jax.numpy as jnp
from jax import lax
from jax.experimental import pallas as pl
from jax.experimental.pallas import tpu as pltpu
from jax.experimental.pallas import tpu_sc as plsc

NC = 2
NS = 16
NW = NC * NS
V = 20
D = 80
B = 64 * 4096
BPW = B // NW
C = 256
NCHUNK = BPW // C
G = C // 16


def _emb_kernel(idx_hbm, table_hbm, out_hbm, table_v, idx_v, rows_v,
                osem0, osem1):
    wid = lax.axis_index("s") * NC + lax.axis_index("c")
    base = wid * BPW
    osems = (osem0, osem1)
    pltpu.sync_copy(table_hbm, table_v)
    iota = lax.iota(jnp.int32, 16)

    def compute_chunk(chunk_start, b):
        pltpu.sync_copy(idx_hbm.at[pl.ds(chunk_start, C)], idx_v.at[b])

        def group_body(g, carry):
            idxv = idx_v[b, pl.ds(g * 16, 16)]
            srcbase = idxv * D
            dstrow = g * 16
            for l in range(16):
                src = srcbase[l]
                dst = (dstrow + l) * D
                for j in range(D // 16):
                    rows_v[b, pl.ds(dst + 16 * j, 16)] = (
                        table_v[pl.ds(src + 16 * j, 16)])
            return carry

        lax.fori_loop(0, G, group_body, 0)

    for b in range(2):
        compute_chunk(base + b * C, b)
        pltpu.async_copy(rows_v.at[b],
                         out_hbm.at[pl.ds((base + b * C) * D, C * D)],
                         osems[b])

    def outer(i, carry):
        for b in range(2):
            cid = 2 * i + b
            pltpu.make_async_copy(rows_v.at[b],
                                  out_hbm.at[pl.ds(base * D, C * D)],
                                  osems[b]).wait()
            compute_chunk(base + cid * C, b)
            pltpu.make_async_copy(rows_v.at[b],
                                  out_hbm.at[pl.ds((base + cid * C) * D, C * D)],
                                  osems[b]).start()
        return carry

    lax.fori_loop(1, NCHUNK // 2, outer, 0)

    for b in range(2):
        pltpu.make_async_copy(rows_v.at[b],
                              out_hbm.at[pl.ds(base * D, C * D)],
                              osems[b]).wait()


@jax.jit
def _emb(idx_flat, table_flat):
    mesh = plsc.VectorSubcoreMesh(core_axis_name="c", subcore_axis_name="s")
    run = pl.kernel(
        _emb_kernel,
        out_type=jax.ShapeDtypeStruct((B * D,), jnp.float32),
        mesh=mesh,
        scratch_types=[
            pltpu.VMEM((V * D,), jnp.float32),
            pltpu.VMEM((2, C), jnp.int32),
            pltpu.VMEM((2, C * D), jnp.float32),
            pltpu.SemaphoreType.DMA,
            pltpu.SemaphoreType.DMA,
        ],
        compiler_params=pltpu.CompilerParams(use_tc_tiling_on_sc=False),
    )
    return run(idx_flat, table_flat)


def kernel(atom_types, table):
    idx_flat = atom_types.reshape(-1).astype(jnp.int32)
    out = _emb(idx_flat, table.reshape(-1))
    return out.reshape(atom_types.shape + (D,))


# speedup vs baseline: 3.2096x; 1.2711x over previous
import jax
import jax.numpy as jnp
from jax import lax
from jax.experimental import pallas as pl
from jax.experimental.pallas import tpu as pltpu
from jax.experimental.pallas import tpu_sc as plsc

NC = 2
NS = 16
NW = NC * NS
V = 20
D = 80
B = 64 * 4096
BPW = B // NW
C = 256
NCHUNK = BPW // C


def _emb_kernel(idx_hbm, table_hbm, out_hbm, table_v, idx_v, rows_v,
                osem0, osem1):
    wid = lax.axis_index("s") * NC + lax.axis_index("c")
    base = wid * BPW
    osems = (osem0, osem1)
    pltpu.sync_copy(table_hbm, table_v)

    def compute_chunk(chunk_start, b):
        pltpu.sync_copy(idx_hbm.at[pl.ds(chunk_start, C)], idx_v.at[b])

        @plsc.parallel_loop(0, C // 16, step=1, unroll=2)
        def group_body(g):
            idxv = idx_v[b, pl.ds(g * 16, 16)]
            srcb = idxv * D
            for l in range(16):
                src = srcb[l]
                dst = (g * 16 + l) * D
                for j in range(D // 16):
                    rows_v[b, pl.ds(dst + 16 * j, 16)] = (
                        table_v[pl.ds(src + 16 * j, 16)])

    for b in range(2):
        compute_chunk(base + b * C, b)
        pltpu.async_copy(rows_v.at[b],
                         out_hbm.at[pl.ds((base + b * C) * D, C * D)],
                         osems[b])

    def outer(i, carry):
        for b in range(2):
            cid = 2 * i + b
            pltpu.make_async_copy(rows_v.at[b],
                                  out_hbm.at[pl.ds(base * D, C * D)],
                                  osems[b]).wait()
            compute_chunk(base + cid * C, b)
            pltpu.make_async_copy(rows_v.at[b],
                                  out_hbm.at[pl.ds((base + cid * C) * D, C * D)],
                                  osems[b]).start()
        return carry

    lax.fori_loop(1, NCHUNK // 2, outer, 0)

    for b in range(2):
        pltpu.make_async_copy(rows_v.at[b],
                              out_hbm.at[pl.ds(base * D, C * D)],
                              osems[b]).wait()


@jax.jit
def _emb(idx_flat, table_flat):
    mesh = plsc.VectorSubcoreMesh(core_axis_name="c", subcore_axis_name="s")
    run = pl.kernel(
        _emb_kernel,
        out_type=jax.ShapeDtypeStruct((B * D,), jnp.float32),
        mesh=mesh,
        scratch_types=[
            pltpu.VMEM((V * D,), jnp.float32),
            pltpu.VMEM((2, C), jnp.int32),
            pltpu.VMEM((2, C * D), jnp.float32),
            pltpu.SemaphoreType.DMA,
            pltpu.SemaphoreType.DMA,
        ],
        compiler_params=pltpu.CompilerParams(use_tc_tiling_on_sc=False),
    )
    return run(idx_flat, table_flat)


def kernel(atom_types, table):
    idx_flat = atom_types.reshape(-1).astype(jnp.int32)
    out = _emb(idx_flat, table.reshape(-1))
    return out.reshape(atom_types.shape + (D,))


# speedup vs baseline: 5.0835x; 1.5838x over previous
import jax
import jax.numpy as jnp
from jax import lax
from jax.experimental import pallas as pl
from jax.experimental.pallas import tpu as pltpu
from jax.experimental.pallas import tpu_sc as plsc

NC = 2
NS = 16
NW = NC * NS
V = 20
D = 80
B = 64 * 4096
BPW = B // NW
C = 256
NCHUNK = BPW // C


def _emb_kernel(idx_hbm, table_hbm, out_hbm, table_v, idx_v, rows_v,
                osem0, osem1):
    wid = lax.axis_index("s") * NC + lax.axis_index("c")
    base = wid * BPW
    osems = (osem0, osem1)
    pltpu.sync_copy(table_hbm, table_v)

    def compute_chunk(cid, b):
        row = 2 * wid + cid // (4096 // C)
        col = (cid % (4096 // C)) * C
        pltpu.sync_copy(idx_hbm.at[row, pl.ds(col, C)], idx_v.at[b])

        @plsc.parallel_loop(0, C // 16, step=1, unroll=2)
        def group_body(g):
            idxv = idx_v[b, pl.ds(g * 16, 16)]
            for l in range(16):
                src = idxv[l]
                dst = g * 16 + l
                for j in range(D // 16):
                    rows_v[b, dst, pl.ds(16 * j, 16)] = (
                        table_v[src, pl.ds(16 * j, 16)])

    for b in range(2):
        compute_chunk(b, b)
        pltpu.async_copy(rows_v.at[b], out_hbm.at[pl.ds(base + b * C, C)],
                         osems[b])

    def outer(i, carry):
        for b in range(2):
            cid = 2 * i + b
            pltpu.make_async_copy(rows_v.at[b], out_hbm.at[pl.ds(base, C)],
                                  osems[b]).wait()
            compute_chunk(cid, b)
            pltpu.make_async_copy(rows_v.at[b],
                                  out_hbm.at[pl.ds(base + cid * C, C)],
                                  osems[b]).start()
        return carry

    lax.fori_loop(1, NCHUNK // 2, outer, 0)

    for b in range(2):
        pltpu.make_async_copy(rows_v.at[b], out_hbm.at[pl.ds(base, C)],
                              osems[b]).wait()


@jax.jit
def _emb(atom_types, table):
    mesh = plsc.VectorSubcoreMesh(core_axis_name="c", subcore_axis_name="s")
    run = pl.kernel(
        _emb_kernel,
        out_type=jax.ShapeDtypeStruct((B, D), jnp.float32),
        mesh=mesh,
        scratch_types=[
            pltpu.VMEM((V, D), jnp.float32),
            pltpu.VMEM((2, C), jnp.int32),
            pltpu.VMEM((2, C, D), jnp.float32),
            pltpu.SemaphoreType.DMA,
            pltpu.SemaphoreType.DMA,
        ],
    )
    return run(atom_types, table)


def kernel(atom_types, table):
    out = _emb(atom_types.astype(jnp.int32), table)
    return out.reshape(atom_types.shape + (D,))
